# baseline (speedup 1.0000x reference)
import functools

import jax
import jax.numpy as jnp
from jax import lax
from jax.experimental import pallas as pl
from jax.experimental.pallas import tpu as pltpu
from jax.experimental.pallas import tpu_sc as plsc

N = 16384
NUM_CLASSES = 1000
NC = 2
NS = 16
NW = NC * NS
RPW = N // NW
R = 32
G = RPW // R
NRING = 3

_mesh = plsc.VectorSubcoreMesh(core_axis_name="c", subcore_axis_name="s")


@functools.partial(
    pl.kernel,
    out_type=jax.ShapeDtypeStruct((N, NUM_CLASSES), jnp.int32),
    mesh=_mesh,
    compiler_params=pltpu.CompilerParams(
        use_tc_tiling_on_sc=False, needs_layout_passes=False
    ),
    scratch_types=[
        pltpu.VMEM((RPW,), jnp.int32),
        pltpu.VMEM((R, NUM_CLASSES), jnp.int32),
        pltpu.VMEM((R, NUM_CLASSES), jnp.int32),
        pltpu.VMEM((R, NUM_CLASSES), jnp.int32),
        pltpu.SemaphoreType.DMA,
        pltpu.SemaphoreType.DMA,
        pltpu.SemaphoreType.DMA,
    ],
)
def _sc_onehot(x_hbm, out_hbm, idx_v, buf0, buf1, buf2, sem0, sem1, sem2):
    bufs = (buf0, buf1, buf2)
    sems = (sem0, sem1, sem2)
    wid = lax.axis_index("s") * NC + lax.axis_index("c")
    base = wid * RPW

    pltpu.sync_copy(x_hbm.at[pl.ds(base, RPW)], idx_v)

    zeros = jnp.zeros((16,), jnp.int32)
    ones = jnp.full((16,), 1, jnp.int32)
    lane = lax.broadcasted_iota(jnp.int32, (16,), 0)

    def _zero_row(r, carry):
        for buf in bufs:
            for c in range(62):
                buf[r, pl.ds(c * 16, 16)] = zeros
            buf[r, pl.ds(NUM_CLASSES - 16, 16)] = zeros
        return carry

    lax.fori_loop(0, R, _zero_row, 0)

    def _scatter(buf, g, val):
        for t in range(R // 16):
            cols = idx_v[pl.ds(g * R + t * 16, 16)]
            plsc.store_scatter(buf, [t * 16 + lane, cols], val)

    for g in range(G):
        b = g % NRING
        if g >= NRING:
            pltpu.make_async_copy(
                bufs[b],
                out_hbm.at[pl.ds(base + (g - NRING) * R, R), :],
                sems[b],
            ).wait()
            _scatter(bufs[b], g - NRING, zeros)
        _scatter(bufs[b], g, ones)
        pltpu.make_async_copy(
            bufs[b],
            out_hbm.at[pl.ds(base + g * R, R), :],
            sems[b],
        ).start()

    for g in range(G - NRING, G):
        b = g % NRING
        pltpu.make_async_copy(
            bufs[b],
            out_hbm.at[pl.ds(base + g * R, R), :],
            sems[b],
        ).wait()


def kernel(x):
    return _sc_onehot(x)


# speedup vs baseline: 1.6183x; 1.6183x over previous
import functools

import jax
import jax.numpy as jnp
from jax import lax
from jax.experimental import pallas as pl
from jax.experimental.pallas import tpu as pltpu
from jax.experimental.pallas import tpu_sc as plsc

N = 16384
NUM_CLASSES = 1000
NC = 2
NS = 16
NW = NC * NS
RPW = N // NW
R = 32
G = RPW // R
NRING = 3

_mesh = plsc.VectorSubcoreMesh(core_axis_name="c", subcore_axis_name="s")


@functools.partial(
    pl.kernel,
    out_type=jax.ShapeDtypeStruct((N, NUM_CLASSES), jnp.int32),
    mesh=_mesh,
    compiler_params=pltpu.CompilerParams(
        use_tc_tiling_on_sc=True, needs_layout_passes=False
    ),
    scratch_types=[
        pltpu.VMEM((RPW,), jnp.int32),
        pltpu.VMEM((R, NUM_CLASSES), jnp.int32),
        pltpu.VMEM((R, NUM_CLASSES), jnp.int32),
        pltpu.VMEM((R, NUM_CLASSES), jnp.int32),
        pltpu.SemaphoreType.DMA,
        pltpu.SemaphoreType.DMA,
        pltpu.SemaphoreType.DMA,
    ],
)
def _sc_onehot(x_hbm, out_hbm, idx_v, buf0, buf1, buf2, sem0, sem1, sem2):
    bufs = (buf0, buf1, buf2)
    sems = (sem0, sem1, sem2)
    wid = lax.axis_index("s") * NC + lax.axis_index("c")
    base = wid * RPW

    pltpu.sync_copy(x_hbm.at[pl.ds(base, RPW)], idx_v)

    zeros = jnp.zeros((16,), jnp.int32)
    ones = jnp.full((16,), 1, jnp.int32)
    lane = lax.broadcasted_iota(jnp.int32, (16,), 0)

    def _zero_row(r, carry):
        for buf in bufs:
            for c in range(62):
                buf[r, pl.ds(c * 16, 16)] = zeros
            buf[r, pl.ds(NUM_CLASSES - 16, 16)] = zeros
        return carry

    lax.fori_loop(0, R, _zero_row, 0)

    def _scatter(buf, g, val):
        for t in range(R // 16):
            cols = idx_v[pl.ds(g * R + t * 16, 16)]
            plsc.store_scatter(buf, [t * 16 + lane, cols], val)

    for g in range(G):
        b = g % NRING
        if g >= NRING:
            pltpu.make_async_copy(
                bufs[b],
                out_hbm.at[pl.ds(base + (g - NRING) * R, R), :],
                sems[b],
            ).wait()
            _scatter(bufs[b], g - NRING, zeros)
        _scatter(bufs[b], g, ones)
        pltpu.make_async_copy(
            bufs[b],
            out_hbm.at[pl.ds(base + g * R, R), :],
            sems[b],
        ).start()

    for g in range(G - NRING, G):
        b = g % NRING
        pltpu.make_async_copy(
            bufs[b],
            out_hbm.at[pl.ds(base + g * R, R), :],
            sems[b],
        ).wait()


def kernel(x):
    return _sc_onehot(x)


# speedup vs baseline: 2.1089x; 1.3032x over previous
import jax
import jax.numpy as jnp
from jax.experimental import pallas as pl
from jax.experimental.pallas import tpu as pltpu

N = 16384
NUM_CLASSES = 1000
BLOCK_ROWS = 512
GRID = N // BLOCK_ROWS
NBUF = 4


def _onehot_block(x_ref, out_ref, scratch_ref, sems):
    i = pl.program_id(0)
    slot = jax.lax.rem(i, NBUF)

    @pl.when(i >= NBUF)
    def _wait_slot():
        pltpu.make_async_copy(
            scratch_ref.at[slot],
            out_ref.at[pl.ds((i - NBUF) * BLOCK_ROWS, BLOCK_ROWS), :],
            sems.at[slot],
        ).wait()

    idx = x_ref[0, 0, :].reshape(BLOCK_ROWS, 1)
    cols = jax.lax.broadcasted_iota(jnp.int32, (BLOCK_ROWS, NUM_CLASSES), 1)
    scratch_ref[slot] = (idx == cols).astype(jnp.int32)

    for j in range(NBUF):
        @pl.when(slot == j)
        def _start(j=j):
            pltpu.make_async_copy(
                scratch_ref.at[j],
                out_ref.at[pl.ds(i * BLOCK_ROWS, BLOCK_ROWS), :],
                sems.at[j],
            ).start(priority=j % 2)

    @pl.when(i == GRID - 1)
    def _drain():
        for j in range(NBUF):
            step = GRID - NBUF + j
            s = step % NBUF
            pltpu.make_async_copy(
                scratch_ref.at[s],
                out_ref.at[pl.ds(step * BLOCK_ROWS, BLOCK_ROWS), :],
                sems.at[s],
            ).wait()


def kernel(x):
    x3 = x.reshape(GRID, 1, BLOCK_ROWS)
    return pl.pallas_call(
        _onehot_block,
        grid=(GRID,),
        in_specs=[pl.BlockSpec((1, 1, BLOCK_ROWS), lambda i: (i, 0, 0))],
        out_specs=pl.BlockSpec(memory_space=pl.ANY),
        out_shape=jax.ShapeDtypeStruct((N, NUM_CLASSES), jnp.int32),
        scratch_shapes=[
            pltpu.VMEM((NBUF, BLOCK_ROWS, NUM_CLASSES), jnp.int32),
            pltpu.SemaphoreType.DMA((NBUF,)),
        ],
    )(x3)
